# Initial kernel scaffold; baseline (speedup 1.0000x reference)
import jax
import jax.numpy as jnp
from jax.experimental import pallas as pl


def kernel(x, w1a1, w1a2, b1a, w1b1, w1b2, b1b, w21, w22, b2, edge_index):
    raise NotImplementedError("write your pallas kernel here")


# speedup vs baseline: 16.6278x; 16.6278x over previous
import functools

import jax
import jax.numpy as jnp
from jax import lax
from jax.experimental import pallas as pl
from jax.experimental.pallas import tpu as pltpu
from jax.experimental.pallas import tpu_sc as plsc

N = 10000
E = 320000
NC = 2
NS = 16
NW = NC * NS
EPB = 128
EPT = 10240
E_PAD = NW * EPT
N_PAD = 10240
RPT = N_PAD // NS
BM = 512


def _make_seg_sum(width):
  n_chunks = EPT // EPB
  mesh = plsc.VectorSubcoreMesh(core_axis_name="c", subcore_axis_name="s")

  @functools.partial(
      pl.kernel,
      out_type=jax.ShapeDtypeStruct((NC, N_PAD, width), jnp.float32),
      mesh=mesh,
      compiler_params=pltpu.CompilerParams(use_tc_tiling_on_sc=False),
      scratch_types=[
          pltpu.VMEM((n_chunks, EPB), jnp.int32),
          pltpu.VMEM((n_chunks, EPB), jnp.int32),
          pltpu.VMEM((EPB, width), jnp.float32),
          pltpu.VMEM_SHARED((N_PAD, width), jnp.float32),
          pltpu.SemaphoreType.DMA,
      ],
  )
  def seg(m_hbm, src_hbm, dst_hbm, zeros_hbm, out_hbm,
          src_v, dst_v, rows_v, acc_sh, sem):
    c = lax.axis_index("c")
    s = lax.axis_index("s")
    wid = c * NS + s
    pltpu.sync_copy(zeros_hbm.at[pl.ds(s * RPT, RPT)],
                    acc_sh.at[pl.ds(s * RPT, RPT)])
    pltpu.sync_copy(src_hbm.at[wid], src_v)
    pltpu.sync_copy(dst_hbm.at[wid], dst_v)
    plsc.subcore_barrier()

    def body(j, carry):
      pltpu.async_copy(m_hbm.at[src_v.at[j]], rows_v, sem).wait()
      pltpu.sync_copy(rows_v, acc_sh.at[dst_v.at[j]], add=True)
      return carry

    lax.fori_loop(0, n_chunks, body, 0)
    plsc.subcore_barrier()
    pltpu.sync_copy(acc_sh.at[pl.ds(s * RPT, RPT)],
                    out_hbm.at[c, pl.ds(s * RPT, RPT)])

  return seg


_seg32 = _make_seg_sum(32)
_seg16 = _make_seg_sum(16)


def _tc1_body(x_ref, w_ref, b_ref, m_ref, s_ref):
  acc = jnp.dot(x_ref[...], w_ref[...], preferred_element_type=jnp.float32)
  m_ref[...] = acc[:, :32]
  s_ref[...] = acc[:, 32:] + b_ref[...]


def _elu(v):
  return jnp.where(v > 0.0, v, jnp.exp(v) - 1.0)


def _tc2_body(a0_ref, a1_ref, s_ref, w21_ref, w22_ref, b2_ref, p_ref, s2_ref):
  z = a0_ref[0] + a1_ref[0] + s_ref[...]
  h = _elu(0.5 * (_elu(z[:, :16]) + _elu(z[:, 16:])))
  p_ref[...] = jnp.dot(h, w21_ref[...], preferred_element_type=jnp.float32)
  s2_ref[...] = (jnp.dot(h, w22_ref[...], preferred_element_type=jnp.float32)
                 + b2_ref[...])


def _tc3_body(a0_ref, a1_ref, s2_ref, o_ref):
  z = a0_ref[0] + a1_ref[0] + s2_ref[...]
  col = lax.broadcasted_iota(jnp.int32, z.shape, 1)
  z = jnp.where(col < 7, z, -1e30)
  z = z - jnp.max(z, axis=1, keepdims=True)
  e = jnp.exp(z)
  o_ref[...] = e / jnp.sum(e, axis=1, keepdims=True)


def kernel(x, w1a1, w1a2, b1a, w1b1, w1b2, b1b, w21, w22, b2, edge_index):
  f32 = jnp.float32
  zrow = jnp.zeros((1, 16), f32)
  k1 = jnp.concatenate([w1a1, zrow, w1b1, zrow], axis=0).reshape(2, 128, 16)
  k1 = jnp.concatenate([k1[0], k1[1]], axis=1)
  k2 = jnp.concatenate([w1a2, zrow, w1b2, zrow], axis=0).reshape(2, 128, 16)
  k2 = jnp.concatenate([k2[0], k2[1]], axis=1)
  w_all = jnp.concatenate([k1, k2], axis=1)
  b_cat = jnp.concatenate([b1a, b1b]).reshape(1, 32)
  zcol = jnp.zeros((16, 9), f32)
  w21p = jnp.concatenate([w21, zcol], axis=1)
  w22p = jnp.concatenate([w22, zcol], axis=1)
  b2p = jnp.concatenate([b2, jnp.zeros((9,), f32)]).reshape(1, 16)

  src = edge_index[0]
  dst = edge_index[1]
  pidx = jnp.arange(E_PAD - E, dtype=jnp.int32)
  src_p = jnp.concatenate([src, pidx % N]).reshape(NW, EPT // EPB, EPB)
  dst_p = jnp.concatenate([dst, N + pidx % (N_PAD - N)]).reshape(
      NW, EPT // EPB, EPB)
  zeros32 = jnp.zeros((N_PAD, 32), f32)
  zeros16 = jnp.zeros((N_PAD, 16), f32)

  nblk = (N + BM - 1) // BM
  m1, s1 = pl.pallas_call(
      _tc1_body,
      grid=(nblk,),
      in_specs=[
          pl.BlockSpec((BM, 128), lambda i: (i, 0)),
          pl.BlockSpec((128, 64), lambda i: (0, 0)),
          pl.BlockSpec((1, 32), lambda i: (0, 0)),
      ],
      out_specs=[
          pl.BlockSpec((BM, 32), lambda i: (i, 0)),
          pl.BlockSpec((BM, 32), lambda i: (i, 0)),
      ],
      out_shape=[
          jax.ShapeDtypeStruct((N, 32), f32),
          jax.ShapeDtypeStruct((N, 32), f32),
      ],
  )(x, w_all, b_cat)

  parts1 = _seg32(m1, src_p, dst_p, zeros32)

  p2, s2 = pl.pallas_call(
      _tc2_body,
      grid=(nblk,),
      in_specs=[
          pl.BlockSpec((1, BM, 32), lambda i: (0, i, 0)),
          pl.BlockSpec((1, BM, 32), lambda i: (1, i, 0)),
          pl.BlockSpec((BM, 32), lambda i: (i, 0)),
          pl.BlockSpec((16, 16), lambda i: (0, 0)),
          pl.BlockSpec((16, 16), lambda i: (0, 0)),
          pl.BlockSpec((1, 16), lambda i: (0, 0)),
      ],
      out_specs=[
          pl.BlockSpec((BM, 16), lambda i: (i, 0)),
          pl.BlockSpec((BM, 16), lambda i: (i, 0)),
      ],
      out_shape=[
          jax.ShapeDtypeStruct((N, 16), f32),
          jax.ShapeDtypeStruct((N, 16), f32),
      ],
  )(parts1, parts1, s1, w21p, w22p, b2p)

  parts2 = _seg16(p2, src_p, dst_p, zeros16)

  o = pl.pallas_call(
      _tc3_body,
      grid=(nblk,),
      in_specs=[
          pl.BlockSpec((1, BM, 16), lambda i: (0, i, 0)),
          pl.BlockSpec((1, BM, 16), lambda i: (1, i, 0)),
          pl.BlockSpec((BM, 16), lambda i: (i, 0)),
      ],
      out_specs=pl.BlockSpec((BM, 16), lambda i: (i, 0)),
      out_shape=jax.ShapeDtypeStruct((N, 16), f32),
  )(parts2, parts2, s2)

  return o[:, :7]


# speedup vs baseline: 28.1591x; 1.6935x over previous
import functools

import jax
import jax.numpy as jnp
from jax import lax
from jax.experimental import pallas as pl
from jax.experimental.pallas import tpu as pltpu
from jax.experimental.pallas import tpu_sc as plsc

N = 10000
E = 320000
NC = 2
NS = 16
NW = NC * NS
EPB = 128
EPT = 10240
E_PAD = NW * EPT
N_PAD = 10240
RPT = N_PAD // NS
BM = 512


NB = 8


def _make_seg_sum(width):
  n_chunks = EPT // EPB
  n_outer = n_chunks // NB - 1
  mesh = plsc.VectorSubcoreMesh(core_axis_name="c", subcore_axis_name="s")

  @functools.partial(
      pl.kernel,
      out_type=jax.ShapeDtypeStruct((NC, N_PAD, width), jnp.float32),
      mesh=mesh,
      compiler_params=pltpu.CompilerParams(use_tc_tiling_on_sc=False),
      scratch_types=[
          pltpu.VMEM((n_chunks, EPB), jnp.int32),
          pltpu.VMEM((n_chunks, EPB), jnp.int32),
          pltpu.VMEM((NB, EPB, width), jnp.float32),
          pltpu.VMEM_SHARED((N_PAD, width), jnp.float32),
          pltpu.SemaphoreType.DMA((NB,)),
      ],
  )
  def seg(m_hbm, src_hbm, dst_hbm, zeros_hbm, out_hbm,
          src_v, dst_v, rows_v, acc_sh, sems):
    c = lax.axis_index("c")
    s = lax.axis_index("s")
    wid = c * NS + s
    pltpu.sync_copy(zeros_hbm.at[pl.ds(s * RPT, RPT)],
                    acc_sh.at[pl.ds(s * RPT, RPT)])
    pltpu.sync_copy(src_hbm.at[wid], src_v)
    pltpu.sync_copy(dst_hbm.at[wid], dst_v)
    plsc.subcore_barrier()

    def gather(j, b):
      return pltpu.make_async_copy(
          m_hbm.at[src_v.at[j]], rows_v.at[b], sems.at[b])

    for b in range(NB):
      gather(b, b).start()

    def body(i, carry):
      j0 = i * NB
      for b in range(NB):
        gather(j0 + b, b).wait()
        pltpu.sync_copy(rows_v.at[b], acc_sh.at[dst_v.at[j0 + b]], add=True)
        gather(j0 + b + NB, b).start()
      return carry

    lax.fori_loop(0, n_outer, body, 0)
    j0 = n_outer * NB
    for b in range(NB):
      gather(j0 + b, b).wait()
      pltpu.sync_copy(rows_v.at[b], acc_sh.at[dst_v.at[j0 + b]], add=True)
    plsc.subcore_barrier()
    pltpu.sync_copy(acc_sh.at[pl.ds(s * RPT, RPT)],
                    out_hbm.at[c, pl.ds(s * RPT, RPT)])

  return seg


_seg32 = _make_seg_sum(32)
_seg16 = _make_seg_sum(16)


def _tc1_body(x_ref, w_ref, b_ref, m_ref, s_ref):
  acc = jnp.dot(x_ref[...], w_ref[...], preferred_element_type=jnp.float32)
  m_ref[...] = acc[:, :32]
  s_ref[...] = acc[:, 32:] + b_ref[...]


def _elu(v):
  return jnp.where(v > 0.0, v, jnp.exp(v) - 1.0)


def _tc2_body(a0_ref, a1_ref, s_ref, w21_ref, w22_ref, b2_ref, p_ref, s2_ref):
  z = a0_ref[0] + a1_ref[0] + s_ref[...]
  h = _elu(0.5 * (_elu(z[:, :16]) + _elu(z[:, 16:])))
  p_ref[...] = jnp.dot(h, w21_ref[...], preferred_element_type=jnp.float32)
  s2_ref[...] = (jnp.dot(h, w22_ref[...], preferred_element_type=jnp.float32)
                 + b2_ref[...])


def _tc3_body(a0_ref, a1_ref, s2_ref, o_ref):
  z = a0_ref[0] + a1_ref[0] + s2_ref[...]
  col = lax.broadcasted_iota(jnp.int32, z.shape, 1)
  z = jnp.where(col < 7, z, -1e30)
  z = z - jnp.max(z, axis=1, keepdims=True)
  e = jnp.exp(z)
  o_ref[...] = e / jnp.sum(e, axis=1, keepdims=True)


def kernel(x, w1a1, w1a2, b1a, w1b1, w1b2, b1b, w21, w22, b2, edge_index):
  f32 = jnp.float32
  zrow = jnp.zeros((1, 16), f32)
  k1 = jnp.concatenate([w1a1, zrow, w1b1, zrow], axis=0).reshape(2, 128, 16)
  k1 = jnp.concatenate([k1[0], k1[1]], axis=1)
  k2 = jnp.concatenate([w1a2, zrow, w1b2, zrow], axis=0).reshape(2, 128, 16)
  k2 = jnp.concatenate([k2[0], k2[1]], axis=1)
  w_all = jnp.concatenate([k1, k2], axis=1)
  b_cat = jnp.concatenate([b1a, b1b]).reshape(1, 32)
  zcol = jnp.zeros((16, 9), f32)
  w21p = jnp.concatenate([w21, zcol], axis=1)
  w22p = jnp.concatenate([w22, zcol], axis=1)
  b2p = jnp.concatenate([b2, jnp.zeros((9,), f32)]).reshape(1, 16)

  src = edge_index[0]
  dst = edge_index[1]
  pidx = jnp.arange(E_PAD - E, dtype=jnp.int32)
  src_p = jnp.concatenate([src, pidx % N]).reshape(NW, EPT // EPB, EPB)
  dst_p = jnp.concatenate([dst, N + pidx % (N_PAD - N)]).reshape(
      NW, EPT // EPB, EPB)
  zeros32 = jnp.zeros((N_PAD, 32), f32)
  zeros16 = jnp.zeros((N_PAD, 16), f32)

  nblk = (N + BM - 1) // BM
  m1, s1 = pl.pallas_call(
      _tc1_body,
      grid=(nblk,),
      in_specs=[
          pl.BlockSpec((BM, 128), lambda i: (i, 0)),
          pl.BlockSpec((128, 64), lambda i: (0, 0)),
          pl.BlockSpec((1, 32), lambda i: (0, 0)),
      ],
      out_specs=[
          pl.BlockSpec((BM, 32), lambda i: (i, 0)),
          pl.BlockSpec((BM, 32), lambda i: (i, 0)),
      ],
      out_shape=[
          jax.ShapeDtypeStruct((N, 32), f32),
          jax.ShapeDtypeStruct((N, 32), f32),
      ],
  )(x, w_all, b_cat)

  parts1 = _seg32(m1, src_p, dst_p, zeros32)

  p2, s2 = pl.pallas_call(
      _tc2_body,
      grid=(nblk,),
      in_specs=[
          pl.BlockSpec((1, BM, 32), lambda i: (0, i, 0)),
          pl.BlockSpec((1, BM, 32), lambda i: (1, i, 0)),
          pl.BlockSpec((BM, 32), lambda i: (i, 0)),
          pl.BlockSpec((16, 16), lambda i: (0, 0)),
          pl.BlockSpec((16, 16), lambda i: (0, 0)),
          pl.BlockSpec((1, 16), lambda i: (0, 0)),
      ],
      out_specs=[
          pl.BlockSpec((BM, 16), lambda i: (i, 0)),
          pl.BlockSpec((BM, 16), lambda i: (i, 0)),
      ],
      out_shape=[
          jax.ShapeDtypeStruct((N, 16), f32),
          jax.ShapeDtypeStruct((N, 16), f32),
      ],
  )(parts1, parts1, s1, w21p, w22p, b2p)

  parts2 = _seg16(p2, src_p, dst_p, zeros16)

  o = pl.pallas_call(
      _tc3_body,
      grid=(nblk,),
      in_specs=[
          pl.BlockSpec((1, BM, 16), lambda i: (0, i, 0)),
          pl.BlockSpec((1, BM, 16), lambda i: (1, i, 0)),
          pl.BlockSpec((BM, 16), lambda i: (i, 0)),
      ],
      out_specs=pl.BlockSpec((BM, 16), lambda i: (i, 0)),
      out_shape=jax.ShapeDtypeStruct((N, 16), f32),
  )(parts2, parts2, s2)

  return o[:, :7]


# speedup vs baseline: 33.7077x; 1.1970x over previous
import functools

import numpy as np
import jax
import jax.numpy as jnp
from jax import lax
from jax.experimental import pallas as pl
from jax.experimental.pallas import tpu as pltpu
from jax.experimental.pallas import tpu_sc as plsc

N = 10000
E = 320000
NC = 2
NS = 16
NW = NC * NS
EPB = 128
EPT = 10240
E_PAD = NW * EPT
N_PAD = 10240
RPT = N_PAD // NS
NB = 8

_PIDX = np.arange(E_PAD - E, dtype=np.int32)
_PAD_SRC = _PIDX % N
_PAD_DST = N + _PIDX % (N_PAD - N)


def _make_seg_sum(width):
  n_chunks = EPT // EPB
  n_outer = n_chunks // NB - 1
  mesh = plsc.VectorSubcoreMesh(core_axis_name="c", subcore_axis_name="s")

  @functools.partial(
      pl.kernel,
      out_type=jax.ShapeDtypeStruct((NC, N_PAD, width), jnp.float32),
      mesh=mesh,
      compiler_params=pltpu.CompilerParams(use_tc_tiling_on_sc=False),
      scratch_types=[
          pltpu.VMEM((n_chunks, EPB), jnp.int32),
          pltpu.VMEM((n_chunks, EPB), jnp.int32),
          pltpu.VMEM((NB, EPB, width), jnp.float32),
          pltpu.VMEM((EPB, width), jnp.float32),
          pltpu.VMEM_SHARED((N_PAD, width), jnp.float32),
          pltpu.SemaphoreType.DMA((NB,)),
      ],
  )
  def seg(m_hbm, src_hbm, dst_hbm, out_hbm,
          src_v, dst_v, rows_v, zero_v, acc_sh, sems):
    c = lax.axis_index("c")
    s = lax.axis_index("s")
    wid = c * NS + s

    def gather(j, b):
      return pltpu.make_async_copy(
          m_hbm.at[src_v.at[j]], rows_v.at[b], sems.at[b])

    pltpu.sync_copy(src_hbm.at[wid], src_v)
    pltpu.sync_copy(dst_hbm.at[wid], dst_v)
    for b in range(NB):
      gather(b, b).start()

    z = jnp.zeros((16,), jnp.float32)
    for r in range(EPB):
      for q in range(width // 16):
        zero_v[r, pl.ds(q * 16, 16)] = z
    for t in range(RPT // EPB):
      pltpu.sync_copy(zero_v, acc_sh.at[pl.ds(s * RPT + t * EPB, EPB)])
    plsc.subcore_barrier()

    def body(i, carry):
      j0 = i * NB
      for b in range(NB):
        gather(j0 + b, b).wait()
        pltpu.sync_copy(rows_v.at[b], acc_sh.at[dst_v.at[j0 + b]], add=True)
        gather(j0 + b + NB, b).start()
      return carry

    lax.fori_loop(0, n_outer, body, 0)
    j0 = n_outer * NB
    for b in range(NB):
      gather(j0 + b, b).wait()
      pltpu.sync_copy(rows_v.at[b], acc_sh.at[dst_v.at[j0 + b]], add=True)
    plsc.subcore_barrier()
    pltpu.sync_copy(acc_sh.at[pl.ds(s * RPT, RPT)],
                    out_hbm.at[c, pl.ds(s * RPT, RPT)])

  return seg


_seg32 = _make_seg_sum(32)
_seg16 = _make_seg_sum(16)


def _tc1_body(x_ref, w_ref, b_ref, m_ref, s_ref):
  acc = jnp.dot(x_ref[...], w_ref[...], preferred_element_type=jnp.float32)
  m_ref[...] = acc[:, :32]
  s_ref[...] = acc[:, 32:] + b_ref[...]


def _elu(v):
  return jnp.where(v > 0.0, v, jnp.exp(v) - 1.0)


def _tc2_body(a_ref, s_ref, w21_ref, w22_ref, b2_ref, p_ref, s2_ref):
  z = a_ref[0, :N] + a_ref[1, :N] + s_ref[...]
  h = _elu(0.5 * (_elu(z[:, :16]) + _elu(z[:, 16:])))
  p_ref[...] = jnp.dot(h, w21_ref[...], preferred_element_type=jnp.float32)
  s2_ref[...] = (jnp.dot(h, w22_ref[...], preferred_element_type=jnp.float32)
                 + b2_ref[...])


def _tc3_body(a_ref, s2_ref, o_ref):
  z = a_ref[0, :N] + a_ref[1, :N] + s2_ref[...]
  col = lax.broadcasted_iota(jnp.int32, z.shape, 1)
  z = jnp.where(col < 7, z, -1e30)
  z = z - jnp.max(z, axis=1, keepdims=True)
  e = jnp.exp(z)
  o_ref[...] = (e / jnp.sum(e, axis=1, keepdims=True))[:, :7]


def kernel(x, w1a1, w1a2, b1a, w1b1, w1b2, b1b, w21, w22, b2, edge_index):
  f32 = jnp.float32
  zrow = jnp.zeros((1, 16), f32)
  k1 = jnp.concatenate([w1a1, zrow, w1b1, zrow], axis=0).reshape(2, 128, 16)
  k1 = jnp.concatenate([k1[0], k1[1]], axis=1)
  k2 = jnp.concatenate([w1a2, zrow, w1b2, zrow], axis=0).reshape(2, 128, 16)
  k2 = jnp.concatenate([k2[0], k2[1]], axis=1)
  w_all = jnp.concatenate([k1, k2], axis=1)
  b_cat = jnp.concatenate([b1a, b1b]).reshape(1, 32)
  zcol = jnp.zeros((16, 9), f32)
  w21p = jnp.concatenate([w21, zcol], axis=1)
  w22p = jnp.concatenate([w22, zcol], axis=1)
  b2p = jnp.concatenate([b2, jnp.zeros((9,), f32)]).reshape(1, 16)

  src_p = jnp.concatenate([edge_index[0], jnp.asarray(_PAD_SRC)]).reshape(
      NW, EPT // EPB, EPB)
  dst_p = jnp.concatenate([edge_index[1], jnp.asarray(_PAD_DST)]).reshape(
      NW, EPT // EPB, EPB)

  full = lambda shape: pl.BlockSpec(shape, lambda: (0,) * len(shape))
  m1, s1 = pl.pallas_call(
      _tc1_body,
      in_specs=[full((N, 128)), full((128, 64)), full((1, 32))],
      out_specs=[full((N, 32)), full((N, 32))],
      out_shape=[jax.ShapeDtypeStruct((N, 32), f32),
                 jax.ShapeDtypeStruct((N, 32), f32)],
  )(x, w_all, b_cat)

  parts1 = _seg32(m1, src_p, dst_p)

  p2, s2 = pl.pallas_call(
      _tc2_body,
      in_specs=[full((NC, N_PAD, 32)), full((N, 32)),
                full((16, 16)), full((16, 16)), full((1, 16))],
      out_specs=[full((N, 16)), full((N, 16))],
      out_shape=[jax.ShapeDtypeStruct((N, 16), f32),
                 jax.ShapeDtypeStruct((N, 16), f32)],
  )(parts1, s1, w21p, w22p, b2p)

  parts2 = _seg16(p2, src_p, dst_p)

  return pl.pallas_call(
      _tc3_body,
      in_specs=[full((NC, N_PAD, 16)), full((N, 16))],
      out_specs=full((N, 7)),
      out_shape=jax.ShapeDtypeStruct((N, 7), f32),
  )(parts2, s2)
